# Initial kernel scaffold; baseline (speedup 1.0000x reference)
import jax
import jax.numpy as jnp
from jax.experimental import pallas as pl


def kernel(x, params):
    raise NotImplementedError("write your pallas kernel here")


# speedup vs baseline: 8.5346x; 8.5346x over previous
import functools

import jax
import jax.numpy as jnp
from jax import lax
from jax.experimental import pallas as pl
from jax.experimental.pallas import tpu as pltpu
from jax.experimental.pallas import tpu_sc as plsc

K = 20
P = 4096
NW = 32
PTS_W = P // NW

_HI = jax.lax.Precision.HIGHEST


def _knn_gidx(x):
    B, N, _ = x.shape
    x3 = x[:, :, 0:3]
    inner = -2.0 * jnp.einsum('bnc,bmc->bnm', x3, x3)
    xx = jnp.sum(x3 * x3, axis=-1, keepdims=True)
    pd = -xx - inner - jnp.swapaxes(xx, 1, 2)
    idx = jax.lax.top_k(pd, K)[1]
    base = (jnp.arange(B, dtype=jnp.int32) * N).reshape(B, 1, 1)
    return (idx + base).reshape(-1).astype(jnp.int32)


@functools.lru_cache(maxsize=None)
def _sc_gather_mean(C):
    R = max(C, 16)
    log2c = C.bit_length() - 1
    log2r = R.bit_length() - 1
    ppc = min(PTS_W, max(8, 2048 // R))
    nch = PTS_W // ppc
    ngrp = (ppc * C) // 16

    mesh = plsc.VectorSubcoreMesh(core_axis_name="c", subcore_axis_name="s")

    @functools.partial(
        pl.kernel, mesh=mesh,
        out_type=jax.ShapeDtypeStruct((P * C,), jnp.float32),
        compiler_params=pltpu.CompilerParams(
            needs_layout_passes=False, use_tc_tiling_on_sc=False),
        scratch_types=[
            pltpu.VMEM((ppc * K,), jnp.int32),
            pltpu.VMEM((ppc * K, R), jnp.float32),
            pltpu.VMEM((ppc * C,), jnp.float32),
            pltpu.SemaphoreType.DMA,
        ],
    )
    def k(table_hbm, gidx_hbm, out_hbm, idx_v, rows_v, out_v, sem):
        wid = lax.axis_index("s") * 2 + lax.axis_index("c")
        base_pt = wid * PTS_W
        iota = jnp.arange(16, dtype=jnp.int32)

        for ch in range(nch):
            pt0 = base_pt + ch * ppc
            pltpu.sync_copy(gidx_hbm.at[pl.ds(pt0 * K, ppc * K)], idx_v)
            pltpu.async_copy(table_hbm.at[idx_v], rows_v, sem).wait()

            def grp(g, _):
                o = g * 16 + iota
                acc = jnp.zeros((16,), jnp.float32)
                if R == C:
                    src0 = o * K
                    for t in range(K):
                        src = src0 + t
                        r = lax.shift_right_logical(src, log2c)
                        cc = lax.bitwise_and(src, C - 1)
                        acc = acc + plsc.load_gather(rows_v, [r, cc])
                else:
                    p = lax.shift_right_logical(o, log2c)
                    f0 = lax.bitwise_and(o, C - 1) * K
                    pk = p * K
                    for t in range(K):
                        f = f0 + t
                        j = lax.shift_right_logical(f, log2c)
                        cc = lax.bitwise_and(f, C - 1)
                        acc = acc + plsc.load_gather(rows_v, [pk + j, cc])
                out_v[pl.ds(g * 16, 16)] = acc * (1.0 / K)
                return _

            lax.fori_loop(0, ngrp, grp, None)
            pltpu.sync_copy(out_v, out_hbm.at[pl.ds(pt0 * C, ppc * C)])

    return k


def _agg(h, gidx):
    C = h.shape[1]
    hp = h if C >= 16 else jnp.pad(h, ((0, 0), (0, 16 - C)))
    m_flat = _sc_gather_mean(C)(hp, gidx)
    return m_flat.reshape(P, C)


def _tc_conv(M, h, p, extras):
    C = h.shape[1]
    Cp = p["W"].shape[0]
    w1t = p["W"][:, :C].T
    w2t = p["W"][:, C:].T
    b = p["b"].reshape(1, Cp)
    ops = [M, h, w1t, w2t, b]
    ex_spec = []
    for r, pp in extras:
        if pp is None:
            ops.append(r)
            ex_spec.append(False)
        else:
            ops.extend([r, pp["W"].T, pp["b"].reshape(1, -1)])
            ex_spec.append(True)

    def body(*refs):
        m_ref, h_ref, w1_ref, w2_ref, b_ref = refs[:5]
        rest = refs[5:-1]
        o_ref = refs[-1]
        hv = h_ref[...]
        y = (jnp.dot(m_ref[...] - hv, w1_ref[...], precision=_HI)
             + jnp.dot(hv, w2_ref[...], precision=_HI) + b_ref[...])
        y = jnp.where(y > 0, y, 0.2 * y)
        i = 0
        for has_proj in ex_spec:
            if has_proj:
                r, wp, bp = rest[i], rest[i + 1], rest[i + 2]
                y = y + jnp.dot(r[...], wp[...], precision=_HI) + bp[...]
                i += 3
            else:
                y = y + rest[i][...]
                i += 1
        o_ref[...] = y

    return pl.pallas_call(
        body, out_shape=jax.ShapeDtypeStruct((P, Cp), jnp.float32))(*ops)


def _conv(h, gidx, p, extras=()):
    M = _agg(h, gidx)
    return _tc_conv(M, h, p, extras)


def kernel(x, params):
    B, N, C0 = x.shape
    gidx = _knn_gidx(x)
    h = x.reshape(P, C0)

    h = _conv(h, gidx, params["entry"])
    for mp in params["modules"]:
        minp = h
        minp_c = h.shape[1]
        for ui, u in enumerate(mp["units"]):
            uinp = h
            gcns = u["gcns"]
            for gi, g in enumerate(gcns):
                extras = []
                if gi == len(gcns) - 1:
                    proj = u["f"] if uinp.shape[1] != g["W"].shape[0] else None
                    extras.append((uinp, proj))
                    if ui == len(mp["units"]) - 1:
                        mproj = mp["f"] if minp_c != g["W"].shape[0] else None
                        extras.append((minp, mproj))
                h = _conv(h, gidx, g, extras)
    out = _conv(h, gidx, params["exit"])
    return out.reshape(B, N, 1)


# speedup vs baseline: 9.2959x; 1.0892x over previous
import functools

import jax
import jax.numpy as jnp
from jax import lax
from jax.experimental import pallas as pl
from jax.experimental.pallas import tpu as pltpu
from jax.experimental.pallas import tpu_sc as plsc

K = 20
P = 4096
NW = 32
PTS_W = P // NW

_HI = jax.lax.Precision.HIGHEST


def _knn_gidx(x):
    B, N, _ = x.shape
    x3 = x[:, :, 0:3]
    inner = -2.0 * jnp.einsum('bnc,bmc->bnm', x3, x3)
    xx = jnp.sum(x3 * x3, axis=-1, keepdims=True)
    pd = -xx - inner - jnp.swapaxes(xx, 1, 2)
    idx = jax.lax.top_k(pd, K)[1]
    base = (jnp.arange(B, dtype=jnp.int32) * N).reshape(B, 1, 1)
    return (idx + base).reshape(-1).astype(jnp.int32)


@functools.lru_cache(maxsize=None)
def _sc_gather_mean(C):
    R = max(C, 16)
    log2c = C.bit_length() - 1
    ppc = min(PTS_W // 2, max(8, 2048 // R))
    nch = PTS_W // ppc
    ngrp = (ppc * C) // 16

    mesh = plsc.VectorSubcoreMesh(core_axis_name="c", subcore_axis_name="s")

    @functools.partial(
        pl.kernel, mesh=mesh,
        out_type=jax.ShapeDtypeStruct((P * C,), jnp.float32),
        compiler_params=pltpu.CompilerParams(
            needs_layout_passes=False, use_tc_tiling_on_sc=False),
        scratch_types=[
            pltpu.VMEM((PTS_W * K,), jnp.int32),
            pltpu.VMEM((ppc * K, R), jnp.float32),
            pltpu.VMEM((ppc * K, R), jnp.float32),
            pltpu.VMEM((ppc * C,), jnp.float32),
            pltpu.VMEM((ppc * C,), jnp.float32),
            pltpu.SemaphoreType.DMA,
            pltpu.SemaphoreType.DMA,
            pltpu.SemaphoreType.DMA,
        ],
    )
    def k(table_hbm, gidx_hbm, out_hbm, idx_v, rows_a, rows_b, out_a, out_b,
          sa, sb, so):
        wid = lax.axis_index("s") * 2 + lax.axis_index("c")
        base_pt = wid * PTS_W
        iota = jnp.arange(16, dtype=jnp.int32)
        lane20 = iota * K

        pltpu.sync_copy(gidx_hbm.at[pl.ds(base_pt * K, PTS_W * K)], idx_v)

        rows = (rows_a, rows_b)
        outs = (out_a, out_b)
        sems = (sa, sb)

        def gather(ch):
            buf = ch % 2
            return pltpu.make_async_copy(
                table_hbm.at[idx_v.at[pl.ds(ch * (ppc * K), ppc * K)]],
                rows[buf], sems[buf])

        def out_copy(ch):
            buf = ch % 2
            return pltpu.make_async_copy(
                outs[buf],
                out_hbm.at[pl.ds((base_pt + ch * ppc) * C, ppc * C)], so)

        gather(0).start()
        for ch in range(nch):
            buf = ch % 2
            if ch + 1 < nch:
                gather(ch + 1).start()
            gather(ch).wait()
            rv = rows[buf]
            ov = outs[buf]
            if ch >= 2:
                out_copy(ch - 2).wait()

            @plsc.parallel_loop(0, ngrp, unroll=2)
            def grp(g):
                acc0 = jnp.zeros((16,), jnp.float32)
                acc1 = jnp.zeros((16,), jnp.float32)
                if R == C:
                    src0 = g * (16 * K) + lane20
                    for t in range(K):
                        src = src0 + t
                        r = lax.shift_right_logical(src, log2c)
                        cc = lax.bitwise_and(src, C - 1)
                        v = plsc.load_gather(rv, [r, cc])
                        if t % 2 == 0:
                            acc0 = acc0 + v
                        else:
                            acc1 = acc1 + v
                else:
                    o = g * 16 + iota
                    p = lax.shift_right_logical(o, log2c)
                    f0 = lax.bitwise_and(o, C - 1) * K
                    pk = p * K
                    for t in range(K):
                        f = f0 + t
                        j = lax.shift_right_logical(f, log2c)
                        cc = lax.bitwise_and(f, C - 1)
                        v = plsc.load_gather(rv, [pk + j, cc])
                        if t % 2 == 0:
                            acc0 = acc0 + v
                        else:
                            acc1 = acc1 + v
                ov[pl.ds(g * 16, 16)] = (acc0 + acc1) * (1.0 / K)

            out_copy(ch).start()

        for ch in (nch - 2, nch - 1):
            if ch >= 0:
                out_copy(ch).wait()

    return k


def _agg(h, gidx):
    C = h.shape[1]
    hp = h if C >= 16 else jnp.pad(h, ((0, 0), (0, 16 - C)))
    m_flat = _sc_gather_mean(C)(hp, gidx)
    return m_flat.reshape(P, C)


def _tc_conv(M, h, p, extras):
    C = h.shape[1]
    Cp = p["W"].shape[0]
    w1t = p["W"][:, :C].T
    w2t = p["W"][:, C:].T
    b = p["b"].reshape(1, Cp)
    ops = [M, h, w1t, w2t, b]
    ex_spec = []
    for r, pp in extras:
        if pp is None:
            ops.append(r)
            ex_spec.append(False)
        else:
            ops.extend([r, pp["W"].T, pp["b"].reshape(1, -1)])
            ex_spec.append(True)

    def body(*refs):
        m_ref, h_ref, w1_ref, w2_ref, b_ref = refs[:5]
        rest = refs[5:-1]
        o_ref = refs[-1]
        hv = h_ref[...]
        y = (jnp.dot(m_ref[...] - hv, w1_ref[...], precision=_HI)
             + jnp.dot(hv, w2_ref[...], precision=_HI) + b_ref[...])
        y = jnp.where(y > 0, y, 0.2 * y)
        i = 0
        for has_proj in ex_spec:
            if has_proj:
                r, wp, bp = rest[i], rest[i + 1], rest[i + 2]
                y = y + jnp.dot(r[...], wp[...], precision=_HI) + bp[...]
                i += 3
            else:
                y = y + rest[i][...]
                i += 1
        o_ref[...] = y

    return pl.pallas_call(
        body, out_shape=jax.ShapeDtypeStruct((P, Cp), jnp.float32))(*ops)


def _conv(h, gidx, p, extras=()):
    M = _agg(h, gidx)
    return _tc_conv(M, h, p, extras)


def kernel(x, params):
    B, N, C0 = x.shape
    gidx = _knn_gidx(x)
    h = x.reshape(P, C0)

    h = _conv(h, gidx, params["entry"])
    for mp in params["modules"]:
        minp = h
        minp_c = h.shape[1]
        for ui, u in enumerate(mp["units"]):
            uinp = h
            gcns = u["gcns"]
            for gi, g in enumerate(gcns):
                extras = []
                if gi == len(gcns) - 1:
                    proj = u["f"] if uinp.shape[1] != g["W"].shape[0] else None
                    extras.append((uinp, proj))
                    if ui == len(mp["units"]) - 1:
                        mproj = mp["f"] if minp_c != g["W"].shape[0] else None
                        extras.append((minp, mproj))
                h = _conv(h, gidx, g, extras)
    out = _conv(h, gidx, params["exit"])
    return out.reshape(B, N, 1)


# speedup vs baseline: 12.9243x; 1.3903x over previous
import functools

import jax
import jax.numpy as jnp
from jax import lax
from jax.experimental import pallas as pl
from jax.experimental.pallas import tpu as pltpu
from jax.experimental.pallas import tpu_sc as plsc

K = 20
P = 4096
NW = 32
PTS_W = P // NW

_HI = jax.lax.Precision.HIGHEST


def _knn_gidx(x):
    B, N, _ = x.shape

    def body(x_ref, o_ref, pd_ref):
        b = pl.program_id(0)
        xv = x_ref[0]
        x3 = xv[:, 0:3]
        xx = jnp.sum(x3 * x3, axis=1, keepdims=True)
        ones = jnp.ones((N, 1), jnp.float32)
        a = jnp.concatenate([2.0 * x3, -xx, ones], axis=1)
        bt = jnp.concatenate([x3, ones, -xx], axis=1)
        pd_ref[...] = jax.lax.dot_general(
            a, bt, (((1,), (1,)), ((), ())), precision=_HI)
        col = jax.lax.broadcasted_iota(jnp.int32, (N, N), 1)
        lane = jax.lax.broadcasted_iota(jnp.int32, (N, 32), 1)
        acc = jnp.zeros((N, 32), jnp.int32)
        base = b * N
        for j in range(K):
            v = pd_ref[...]
            m = jnp.max(v, axis=1, keepdims=True)
            am = jnp.min(jnp.where(v == m, col, N), axis=1, keepdims=True)
            pd_ref[...] = jnp.where(col == am, -jnp.inf, v)
            acc = jnp.where(lane == j, am + base, acc)
        o_ref[0] = acc

    out = pl.pallas_call(
        body,
        grid=(B,),
        in_specs=[pl.BlockSpec((1, N, 4), lambda b: (b, 0, 0))],
        out_specs=pl.BlockSpec((1, N, 32), lambda b: (b, 0, 0)),
        out_shape=jax.ShapeDtypeStruct((B, N, 32), jnp.int32),
        scratch_shapes=[pltpu.VMEM((N, N), jnp.float32)],
    )(x)
    return out[:, :, :K].reshape(-1)


@functools.lru_cache(maxsize=None)
def _sc_gather_mean(C):
    R = max(C, 16)
    log2c = C.bit_length() - 1
    ppc = min(PTS_W // 2, max(8, 2048 // R))
    nch = PTS_W // ppc
    ngrp = (ppc * C) // 16

    mesh = plsc.VectorSubcoreMesh(core_axis_name="c", subcore_axis_name="s")

    @functools.partial(
        pl.kernel, mesh=mesh,
        out_type=jax.ShapeDtypeStruct((P * C,), jnp.float32),
        compiler_params=pltpu.CompilerParams(
            needs_layout_passes=False, use_tc_tiling_on_sc=False),
        scratch_types=[
            pltpu.VMEM((PTS_W * K,), jnp.int32),
            pltpu.VMEM((ppc * K, R), jnp.float32),
            pltpu.VMEM((ppc * K, R), jnp.float32),
            pltpu.VMEM((ppc * C,), jnp.float32),
            pltpu.VMEM((ppc * C,), jnp.float32),
            pltpu.SemaphoreType.DMA,
            pltpu.SemaphoreType.DMA,
            pltpu.SemaphoreType.DMA,
        ],
    )
    def k(table_hbm, gidx_hbm, out_hbm, idx_v, rows_a, rows_b, out_a, out_b,
          sa, sb, so):
        wid = lax.axis_index("s") * 2 + lax.axis_index("c")
        base_pt = wid * PTS_W
        iota = jnp.arange(16, dtype=jnp.int32)
        lane20 = iota * K

        pltpu.sync_copy(gidx_hbm.at[pl.ds(base_pt * K, PTS_W * K)], idx_v)

        rows = (rows_a, rows_b)
        outs = (out_a, out_b)
        sems = (sa, sb)

        def gather(ch):
            buf = ch % 2
            return pltpu.make_async_copy(
                table_hbm.at[idx_v.at[pl.ds(ch * (ppc * K), ppc * K)]],
                rows[buf], sems[buf])

        def out_copy(ch):
            buf = ch % 2
            return pltpu.make_async_copy(
                outs[buf],
                out_hbm.at[pl.ds((base_pt + ch * ppc) * C, ppc * C)], so)

        gather(0).start()
        for ch in range(nch):
            buf = ch % 2
            if ch + 1 < nch:
                gather(ch + 1).start()
            gather(ch).wait()
            rv = rows[buf]
            ov = outs[buf]
            if ch >= 2:
                out_copy(ch - 2).wait()

            @plsc.parallel_loop(0, ngrp, unroll=2)
            def grp(g):
                acc0 = jnp.zeros((16,), jnp.float32)
                acc1 = jnp.zeros((16,), jnp.float32)
                if R == C:
                    src0 = g * (16 * K) + lane20
                    for t in range(K):
                        src = src0 + t
                        r = lax.shift_right_logical(src, log2c)
                        cc = lax.bitwise_and(src, C - 1)
                        v = plsc.load_gather(rv, [r, cc])
                        if t % 2 == 0:
                            acc0 = acc0 + v
                        else:
                            acc1 = acc1 + v
                else:
                    o = g * 16 + iota
                    p = lax.shift_right_logical(o, log2c)
                    f0 = lax.bitwise_and(o, C - 1) * K
                    pk = p * K
                    for t in range(K):
                        f = f0 + t
                        j = lax.shift_right_logical(f, log2c)
                        cc = lax.bitwise_and(f, C - 1)
                        v = plsc.load_gather(rv, [pk + j, cc])
                        if t % 2 == 0:
                            acc0 = acc0 + v
                        else:
                            acc1 = acc1 + v
                ov[pl.ds(g * 16, 16)] = (acc0 + acc1) * (1.0 / K)

            out_copy(ch).start()

        for ch in (nch - 2, nch - 1):
            if ch >= 0:
                out_copy(ch).wait()

    return k


def _agg(h, gidx):
    C = h.shape[1]
    hp = h if C >= 16 else jnp.pad(h, ((0, 0), (0, 16 - C)))
    m_flat = _sc_gather_mean(C)(hp, gidx)
    return m_flat.reshape(P, C)


def _tc_conv(M, h, p, extras):
    C = h.shape[1]
    Cp = p["W"].shape[0]
    w1t = p["W"][:, :C].T
    w2t = p["W"][:, C:].T
    b = p["b"].reshape(1, Cp)
    ops = [M, h, w1t, w2t, b]
    ex_spec = []
    for r, pp in extras:
        if pp is None:
            ops.append(r)
            ex_spec.append(False)
        else:
            ops.extend([r, pp["W"].T, pp["b"].reshape(1, -1)])
            ex_spec.append(True)

    def body(*refs):
        m_ref, h_ref, w1_ref, w2_ref, b_ref = refs[:5]
        rest = refs[5:-1]
        o_ref = refs[-1]
        hv = h_ref[...]
        y = (jnp.dot(m_ref[...] - hv, w1_ref[...], precision=_HI)
             + jnp.dot(hv, w2_ref[...], precision=_HI) + b_ref[...])
        y = jnp.where(y > 0, y, 0.2 * y)
        i = 0
        for has_proj in ex_spec:
            if has_proj:
                r, wp, bp = rest[i], rest[i + 1], rest[i + 2]
                y = y + jnp.dot(r[...], wp[...], precision=_HI) + bp[...]
                i += 3
            else:
                y = y + rest[i][...]
                i += 1
        o_ref[...] = y

    return pl.pallas_call(
        body, out_shape=jax.ShapeDtypeStruct((P, Cp), jnp.float32))(*ops)


def _conv(h, gidx, p, extras=()):
    M = _agg(h, gidx)
    return _tc_conv(M, h, p, extras)


def kernel(x, params):
    B, N, C0 = x.shape
    gidx = _knn_gidx(x)
    h = x.reshape(P, C0)

    h = _conv(h, gidx, params["entry"])
    for mp in params["modules"]:
        minp = h
        minp_c = h.shape[1]
        for ui, u in enumerate(mp["units"]):
            uinp = h
            gcns = u["gcns"]
            for gi, g in enumerate(gcns):
                extras = []
                if gi == len(gcns) - 1:
                    proj = u["f"] if uinp.shape[1] != g["W"].shape[0] else None
                    extras.append((uinp, proj))
                    if ui == len(mp["units"]) - 1:
                        mproj = mp["f"] if minp_c != g["W"].shape[0] else None
                        extras.append((minp, mproj))
                h = _conv(h, gidx, g, extras)
    out = _conv(h, gidx, params["exit"])
    return out.reshape(B, N, 1)


# speedup vs baseline: 12.9978x; 1.0057x over previous
import functools

import jax
import jax.numpy as jnp
from jax import lax
from jax.experimental import pallas as pl
from jax.experimental.pallas import tpu as pltpu
from jax.experimental.pallas import tpu_sc as plsc

K = 20
P = 4096
NW = 32
PTS_W = P // NW

_HI = jax.lax.Precision.HIGHEST


def _knn_gidx(x):
    B, N, _ = x.shape

    def body(x_ref, o_ref, pd_ref):
        b = pl.program_id(0)
        xv = x_ref[0]
        x3 = xv[:, 0:3]
        xx = jnp.sum(x3 * x3, axis=1, keepdims=True)
        g = jax.lax.dot_general(x3, x3, (((1,), (1,)), ((), ())))
        inner = -2.0 * g
        xxt = jax.lax.dot_general(
            jnp.ones((1, 1), jnp.float32), xx,
            (((1,), (1,)), ((), ())), precision=_HI)
        pd_ref[...] = (-xx) - inner - xxt
        col = jax.lax.broadcasted_iota(jnp.int32, (N, N), 1)
        lane = jax.lax.broadcasted_iota(jnp.int32, (N, 32), 1)
        acc = jnp.zeros((N, 32), jnp.int32)
        base = b * N
        for j in range(K):
            v = pd_ref[...]
            m = jnp.max(v, axis=1, keepdims=True)
            am = jnp.min(jnp.where(v == m, col, N), axis=1, keepdims=True)
            pd_ref[...] = jnp.where(col == am, -jnp.inf, v)
            acc = jnp.where(lane == j, am + base, acc)
        o_ref[0] = acc

    out = pl.pallas_call(
        body,
        grid=(B,),
        in_specs=[pl.BlockSpec((1, N, 4), lambda b: (b, 0, 0))],
        out_specs=pl.BlockSpec((1, N, 32), lambda b: (b, 0, 0)),
        out_shape=jax.ShapeDtypeStruct((B, N, 32), jnp.int32),
        scratch_shapes=[pltpu.VMEM((N, N), jnp.float32)],
    )(x)
    return out[:, :, :K].reshape(-1)


@functools.lru_cache(maxsize=None)
def _sc_gather_mean(C):
    R = max(C, 16)
    log2c = C.bit_length() - 1
    ppc = min(PTS_W // 2, max(8, 2048 // R))
    nch = PTS_W // ppc
    ngrp = (ppc * C) // 16

    mesh = plsc.VectorSubcoreMesh(core_axis_name="c", subcore_axis_name="s")

    @functools.partial(
        pl.kernel, mesh=mesh,
        out_type=jax.ShapeDtypeStruct((P * C,), jnp.float32),
        compiler_params=pltpu.CompilerParams(
            needs_layout_passes=False, use_tc_tiling_on_sc=False),
        scratch_types=[
            pltpu.VMEM((PTS_W * K,), jnp.int32),
            pltpu.VMEM((ppc * K, R), jnp.float32),
            pltpu.VMEM((ppc * K, R), jnp.float32),
            pltpu.VMEM((ppc * C,), jnp.float32),
            pltpu.VMEM((ppc * C,), jnp.float32),
            pltpu.SemaphoreType.DMA,
            pltpu.SemaphoreType.DMA,
            pltpu.SemaphoreType.DMA,
        ],
    )
    def k(table_hbm, gidx_hbm, out_hbm, idx_v, rows_a, rows_b, out_a, out_b,
          sa, sb, so):
        wid = lax.axis_index("s") * 2 + lax.axis_index("c")
        base_pt = wid * PTS_W
        iota = jnp.arange(16, dtype=jnp.int32)
        lane20 = iota * K

        pltpu.sync_copy(gidx_hbm.at[pl.ds(base_pt * K, PTS_W * K)], idx_v)

        rows = (rows_a, rows_b)
        outs = (out_a, out_b)
        sems = (sa, sb)

        def gather(ch):
            buf = ch % 2
            return pltpu.make_async_copy(
                table_hbm.at[idx_v.at[pl.ds(ch * (ppc * K), ppc * K)]],
                rows[buf], sems[buf])

        def out_copy(ch):
            buf = ch % 2
            return pltpu.make_async_copy(
                outs[buf],
                out_hbm.at[pl.ds((base_pt + ch * ppc) * C, ppc * C)], so)

        gather(0).start()
        for ch in range(nch):
            buf = ch % 2
            if ch + 1 < nch:
                gather(ch + 1).start()
            gather(ch).wait()
            rv = rows[buf]
            ov = outs[buf]
            if ch >= 2:
                out_copy(ch - 2).wait()

            @plsc.parallel_loop(0, ngrp, unroll=2)
            def grp(g):
                acc0 = jnp.zeros((16,), jnp.float32)
                acc1 = jnp.zeros((16,), jnp.float32)
                if R == C:
                    src0 = g * (16 * K) + lane20
                    for t in range(K):
                        src = src0 + t
                        r = lax.shift_right_logical(src, log2c)
                        cc = lax.bitwise_and(src, C - 1)
                        v = plsc.load_gather(rv, [r, cc])
                        if t % 2 == 0:
                            acc0 = acc0 + v
                        else:
                            acc1 = acc1 + v
                else:
                    o = g * 16 + iota
                    p = lax.shift_right_logical(o, log2c)
                    f0 = lax.bitwise_and(o, C - 1) * K
                    pk = p * K
                    for t in range(K):
                        f = f0 + t
                        j = lax.shift_right_logical(f, log2c)
                        cc = lax.bitwise_and(f, C - 1)
                        v = plsc.load_gather(rv, [pk + j, cc])
                        if t % 2 == 0:
                            acc0 = acc0 + v
                        else:
                            acc1 = acc1 + v
                ov[pl.ds(g * 16, 16)] = (acc0 + acc1) * (1.0 / K)

            out_copy(ch).start()

        for ch in (nch - 2, nch - 1):
            if ch >= 0:
                out_copy(ch).wait()

    return k


def _agg(h, gidx):
    C = h.shape[1]
    hp = h if C >= 16 else jnp.pad(h, ((0, 0), (0, 16 - C)))
    m_flat = _sc_gather_mean(C)(hp, gidx)
    return m_flat.reshape(P, C)


def _tc_conv(M, h, p, extras):
    C = h.shape[1]
    Cp = p["W"].shape[0]
    w1t = p["W"][:, :C].T
    w2t = p["W"][:, C:].T
    b = p["b"].reshape(1, Cp)
    ops = [M, h, w1t, w2t, b]
    ex_spec = []
    for r, pp in extras:
        if pp is None:
            ops.append(r)
            ex_spec.append(False)
        else:
            ops.extend([r, pp["W"].T, pp["b"].reshape(1, -1)])
            ex_spec.append(True)

    def body(*refs):
        m_ref, h_ref, w1_ref, w2_ref, b_ref = refs[:5]
        rest = refs[5:-1]
        o_ref = refs[-1]
        hv = h_ref[...]
        y = (jnp.dot(m_ref[...] - hv, w1_ref[...], precision=_HI)
             + jnp.dot(hv, w2_ref[...], precision=_HI) + b_ref[...])
        y = jnp.where(y > 0, y, 0.2 * y)
        i = 0
        for has_proj in ex_spec:
            if has_proj:
                r, wp, bp = rest[i], rest[i + 1], rest[i + 2]
                y = y + jnp.dot(r[...], wp[...], precision=_HI) + bp[...]
                i += 3
            else:
                y = y + rest[i][...]
                i += 1
        o_ref[...] = y

    return pl.pallas_call(
        body, out_shape=jax.ShapeDtypeStruct((P, Cp), jnp.float32))(*ops)


def _conv(h, gidx, p, extras=()):
    M = _agg(h, gidx)
    return _tc_conv(M, h, p, extras)


def kernel(x, params):
    B, N, C0 = x.shape
    gidx = _knn_gidx(x)
    h = x.reshape(P, C0)

    h = _conv(h, gidx, params["entry"])
    for mp in params["modules"]:
        minp = h
        minp_c = h.shape[1]
        for ui, u in enumerate(mp["units"]):
            uinp = h
            gcns = u["gcns"]
            for gi, g in enumerate(gcns):
                extras = []
                if gi == len(gcns) - 1:
                    proj = u["f"] if uinp.shape[1] != g["W"].shape[0] else None
                    extras.append((uinp, proj))
                    if ui == len(mp["units"]) - 1:
                        mproj = mp["f"] if minp_c != g["W"].shape[0] else None
                        extras.append((minp, mproj))
                h = _conv(h, gidx, g, extras)
    out = _conv(h, gidx, params["exit"])
    return out.reshape(B, N, 1)
